# Initial kernel scaffold; baseline (speedup 1.0000x reference)
import jax
import jax.numpy as jnp
from jax.experimental import pallas as pl


def kernel(imgs, pids, camids, g_feats, g_pids, g_camids):
    raise NotImplementedError("write your pallas kernel here")


# speedup vs baseline: 2.3203x; 2.3203x over previous
import functools

import jax
import jax.numpy as jnp
from jax.experimental import pallas as pl
from jax.experimental.pallas import tpu as pltpu

_TOPK = 10
_CHUNK = 2048
_NEGF = float(-1e30)
_BIGI = int(2**30)


def _matcher_body(q_ref, g_ref, gp_ref, gc_ref, p_ref, c_ref,
                  osim_ref, oidx_ref, qs_ref, rv_ref, ri_ref, *, K, nchunks):
    i = pl.program_id(0)
    Q = q_ref.shape[0]

    @pl.when(i == 0)
    def _init():
        x = q_ref[...]
        ss = jnp.sum(x * x, axis=1, keepdims=True)
        qs_ref[...] = x / jnp.maximum(jnp.sqrt(ss), 1e-12)
        rv_ref[...] = jnp.full(rv_ref.shape, _NEGF, jnp.float32)
        ri_ref[...] = jnp.full(ri_ref.shape, _BIGI, jnp.int32)

    g = g_ref[...]
    gss = jnp.sum(g * g, axis=1, keepdims=True)
    gn = g / jnp.maximum(jnp.sqrt(gss), 1e-12)

    sim = jax.lax.dot_general(
        qs_ref[...], gn, (((1,), (1,)), ((), ())),
        preferred_element_type=jnp.float32)

    gp = gp_ref[...]
    gc = gc_ref[...]
    mask = (p_ref[...] == gp) & (c_ref[...] == gc)
    sim = jnp.where(mask, jnp.float32(-1.0), sim)

    colid = i * _CHUNK + jax.lax.broadcasted_iota(jnp.int32, (1, _CHUNK), 1)
    sim = jnp.where(colid < K, sim, _NEGF)

    cA = sim
    cB = rv_ref[...]
    iB = ri_ref[...]
    vals, idxs = [], []
    for _ in range(_TOPK):
        m = jnp.maximum(jnp.max(cA, axis=1, keepdims=True),
                        jnp.max(cB, axis=1, keepdims=True))
        jA = jnp.min(jnp.where(cA == m, colid, _BIGI), axis=1, keepdims=True)
        jB = jnp.min(jnp.where(cB == m, iB, _BIGI), axis=1, keepdims=True)
        j = jnp.minimum(jA, jB)
        vals.append(m)
        idxs.append(j)
        cA = jnp.where(colid == j, _NEGF, cA)
        cB = jnp.where(iB == j, _NEGF, cB)
    rv = jnp.concatenate(vals, axis=1)
    ri = jnp.concatenate(idxs, axis=1)
    rv_ref[:, :_TOPK] = rv
    ri_ref[:, :_TOPK] = ri

    @pl.when(i == nchunks - 1)
    def _fin():
        osim_ref[...] = rv
        oidx_ref[...] = ri


def _topk_sim(q, g_feats, g_pids2, g_camids2, pids2, camids2):
    Q, D = q.shape
    K = g_feats.shape[0]
    nchunks = pl.cdiv(K, _CHUNK)
    RW = 128
    body = functools.partial(_matcher_body, K=K, nchunks=nchunks)
    return pl.pallas_call(
        body,
        grid=(nchunks,),
        in_specs=[
            pl.BlockSpec((Q, D), lambda i: (0, 0)),
            pl.BlockSpec((_CHUNK, D), lambda i: (i, 0)),
            pl.BlockSpec((1, _CHUNK), lambda i: (0, i)),
            pl.BlockSpec((1, _CHUNK), lambda i: (0, i)),
            pl.BlockSpec((Q, 1), lambda i: (0, 0)),
            pl.BlockSpec((Q, 1), lambda i: (0, 0)),
        ],
        out_specs=[
            pl.BlockSpec((Q, _TOPK), lambda i: (0, 0)),
            pl.BlockSpec((Q, _TOPK), lambda i: (0, 0)),
        ],
        out_shape=[
            jax.ShapeDtypeStruct((Q, _TOPK), jnp.float32),
            jax.ShapeDtypeStruct((Q, _TOPK), jnp.int32),
        ],
        scratch_shapes=[
            pltpu.VMEM((Q, D), jnp.float32),
            pltpu.VMEM((Q, RW), jnp.float32),
            pltpu.VMEM((Q, RW), jnp.int32),
        ],
        compiler_params=pltpu.CompilerParams(
            dimension_semantics=("arbitrary",)),
    )(q, g_feats, g_pids2, g_camids2, pids2, camids2)


def kernel(imgs, pids, camids, g_feats, g_pids, g_camids):
    Q = imgs.shape[0]
    topk_sim, topk_index = _topk_sim(
        imgs, g_feats,
        g_pids.reshape(1, -1), g_camids.reshape(1, -1),
        pids.reshape(Q, 1), camids.reshape(Q, 1))
    match_pids = g_pids[topk_index]
    match_camids = g_camids[topk_index]
    matches = pids[:, None] == match_pids
    return (matches, topk_sim, match_pids, match_camids, topk_index)
